# baseline (speedup 1.0000x reference)
import functools

import jax
import jax.numpy as jnp
from jax import lax
from jax.experimental import pallas as pl
from jax.experimental.pallas import tpu as pltpu
from jax.experimental.pallas import tpu_sc as plsc

_info = plsc.get_sparse_core_info()
_NC, _NS, _NL = _info.num_cores, _info.num_subcores, _info.num_lanes
_NW = _NC * _NS


@functools.lru_cache(maxsize=None)
def _build(n_idx: int, n_rows: int):
    assert n_idx % (_NW * _NL) == 0
    bpw = n_idx // _NW

    mesh = plsc.VectorSubcoreMesh(core_axis_name="c", subcore_axis_name="s")

    @functools.partial(
        pl.kernel,
        mesh=mesh,
        compiler_params=pltpu.CompilerParams(needs_layout_passes=False),
        out_type=jax.ShapeDtypeStruct((n_idx,), jnp.float32),
        scratch_types=[
            pltpu.VMEM((bpw,), jnp.int32),
            pltpu.VMEM((bpw,), jnp.float32),
            pltpu.VMEM((n_rows,), jnp.float32),
        ],
    )
    def gather_kernel(idx_hbm, tab_hbm, out_hbm, idx_v, out_v, tab_v):
        wid = lax.axis_index("s") * _NC + lax.axis_index("c")
        base = wid * bpw
        pltpu.sync_copy(idx_hbm.at[pl.ds(base, bpw)], idx_v)
        pltpu.sync_copy(tab_hbm, tab_v)

        def step(i, carry):
            o = i * _NL
            out_v[pl.ds(o, _NL)] = plsc.load_gather(
                tab_v, [idx_v[pl.ds(o, _NL)]]
            )
            return carry

        lax.fori_loop(0, bpw // _NL, step, 0, unroll=8)
        pltpu.sync_copy(out_v, out_hbm.at[pl.ds(base, bpw)])

    return gather_kernel


def kernel(input, table):
    b, h = input.shape
    idx = input.reshape(-1)
    tab = table.reshape(-1)
    out = _build(idx.shape[0], tab.shape[0])(idx, tab)
    return out.reshape(b, h, 1)


# speedup vs baseline: 1.4089x; 1.4089x over previous
import functools

import jax
import jax.numpy as jnp
from jax import lax
from jax.experimental import pallas as pl
from jax.experimental.pallas import tpu as pltpu
from jax.experimental.pallas import tpu_sc as plsc

_info = plsc.get_sparse_core_info()
_NC, _NS = _info.num_cores, _info.num_subcores
_NW = _NC * _NS


@functools.lru_cache(maxsize=None)
def _build(b: int, h: int, n_rows: int):
    assert b % _NW == 0
    rpw = b // _NW

    mesh = plsc.VectorSubcoreMesh(core_axis_name="c", subcore_axis_name="s")

    hp = h

    @functools.partial(
        pl.kernel,
        mesh=mesh,
        compiler_params=pltpu.CompilerParams(needs_layout_passes=False),
        out_type=jax.ShapeDtypeStruct((b, h), jnp.float32),
        scratch_types=[
            pltpu.VMEM((rpw, hp), jnp.int32),
            pltpu.VMEM((rpw, hp), jnp.float32),
            pltpu.VMEM_SHARED((n_rows,), jnp.float32),
            pltpu.SemaphoreType.DMA,
        ],
    )
    def gather_kernel(idx_hbm, tab_hbm, out_hbm, idx_v, rows_v, tab_sh, sem):
        wid = lax.axis_index("s") * _NC + lax.axis_index("c")
        base = wid * rpw

        @pl.when(lax.axis_index("s") == 0)
        def _stage():
            pltpu.sync_copy(tab_hbm, tab_sh)

        pltpu.sync_copy(idx_hbm.at[pl.ds(base, rpw)], idx_v)
        plsc.subcore_barrier()

        k = 8

        def step(g, carry):
            j0 = g * k
            copies = [
                pltpu.async_copy(tab_sh.at[idx_v.at[j0 + j]],
                                 rows_v.at[j0 + j], sem)
                for j in range(k)
            ]
            for c in copies:
                c.wait()
            return carry

        lax.fori_loop(0, rpw // k, step, 0, unroll=False)
        pltpu.sync_copy(rows_v, out_hbm.at[pl.ds(base, rpw)])

    return gather_kernel


def kernel(input, table):
    b, h = input.shape
    tab = table.reshape(-1)
    out = _build(b, h, tab.shape[0])(input, tab)
    return out.reshape(b, h, 1)


# speedup vs baseline: 1.6031x; 1.1378x over previous
import functools

import jax
import jax.numpy as jnp
from jax import lax
from jax.experimental import pallas as pl
from jax.experimental.pallas import tpu as pltpu
from jax.experimental.pallas import tpu_sc as plsc

_info = plsc.get_sparse_core_info()
_NC, _NS = _info.num_cores, _info.num_subcores
_NW = _NC * _NS

_K = 10


@functools.lru_cache(maxsize=None)
def _build(h: int, b: int, n_rows: int):
    assert b % (_NW * 8) == 0 and h % _K == 0
    cpw = b // _NW

    mesh = plsc.VectorSubcoreMesh(core_axis_name="c", subcore_axis_name="s")

    @functools.partial(
        pl.kernel,
        mesh=mesh,
        compiler_params=pltpu.CompilerParams(needs_layout_passes=False),
        out_type=jax.ShapeDtypeStruct((h, b), jnp.float32),
        scratch_types=[
            pltpu.VMEM((h, cpw), jnp.int32),
            pltpu.VMEM((h, cpw), jnp.float32),
            pltpu.VMEM_SHARED((n_rows,), jnp.float32),
            pltpu.SemaphoreType.DMA,
        ],
    )
    def gather_kernel(idx_hbm, tab_hbm, out_hbm, idx_v, rows_v, tab_sh,
                      sem_g):
        wid = lax.axis_index("s") * _NC + lax.axis_index("c")
        cb = wid * cpw

        @pl.when(lax.axis_index("s") == 0)
        def _stage():
            pltpu.sync_copy(tab_hbm, tab_sh)

        pltpu.sync_copy(idx_hbm.at[:, pl.ds(cb, cpw)], idx_v)
        plsc.subcore_barrier()

        def step(g, carry):
            j0 = g * _K
            gathers = [
                pltpu.async_copy(tab_sh.at[idx_v.at[j0 + j]],
                                 rows_v.at[j0 + j], sem_g)
                for j in range(_K)
            ]
            for c in gathers:
                c.wait()
            return carry

        lax.fori_loop(0, h // _K, step, 0, unroll=False)
        pltpu.sync_copy(rows_v, out_hbm.at[:, pl.ds(cb, cpw)])

    return gather_kernel


def kernel(input, table):
    b, h = input.shape
    idx_t = jnp.swapaxes(input, 0, 1)
    tab = jnp.squeeze(table, 1)
    out_t = _build(h, b, tab.shape[0])(idx_t, tab)
    return jnp.swapaxes(out_t, 0, 1)[..., None]


# speedup vs baseline: 1.6628x; 1.0372x over previous
import functools

import jax
import jax.numpy as jnp
from jax import lax
from jax.experimental import pallas as pl
from jax.experimental.pallas import tpu as pltpu
from jax.experimental.pallas import tpu_sc as plsc

_info = plsc.get_sparse_core_info()
_NC, _NS = _info.num_cores, _info.num_subcores
_NW = _NC * _NS

_K = 25


@functools.lru_cache(maxsize=None)
def _build(h: int, b: int, n_rows: int):
    assert b % (_NW * 8) == 0 and h % _K == 0
    cpw = b // _NW

    mesh = plsc.VectorSubcoreMesh(core_axis_name="c", subcore_axis_name="s")

    @functools.partial(
        pl.kernel,
        mesh=mesh,
        compiler_params=pltpu.CompilerParams(needs_layout_passes=False),
        out_type=jax.ShapeDtypeStruct((h, b), jnp.float32),
        scratch_types=[
            pltpu.VMEM((h, cpw), jnp.int32),
            pltpu.VMEM((h, cpw), jnp.float32),
            pltpu.VMEM_SHARED((n_rows,), jnp.float32),
            pltpu.SemaphoreType.DMA,
        ],
    )
    def gather_kernel(idx_hbm, tab_hbm, out_hbm, idx_v, rows_v, tab_sh,
                      sem_g):
        wid = lax.axis_index("s") * _NC + lax.axis_index("c")
        cb = wid * cpw

        @pl.when(lax.axis_index("s") == 0)
        def _stage():
            pltpu.sync_copy(tab_hbm, tab_sh)

        pltpu.sync_copy(idx_hbm.at[:, pl.ds(cb, cpw)], idx_v)
        plsc.subcore_barrier()

        def step(g, carry):
            j0 = g * _K
            gathers = [
                pltpu.async_copy(tab_sh.at[idx_v.at[j0 + j]],
                                 rows_v.at[j0 + j], sem_g)
                for j in range(_K)
            ]
            for c in gathers:
                c.wait()
            return carry

        lax.fori_loop(0, h // _K, step, 0, unroll=False)
        pltpu.sync_copy(rows_v, out_hbm.at[:, pl.ds(cb, cpw)])

    return gather_kernel


def kernel(input, table):
    b, h = input.shape
    idx_t = jnp.swapaxes(input, 0, 1)
    tab = jnp.squeeze(table, 1)
    out_t = _build(h, b, tab.shape[0])(idx_t, tab)
    return jnp.swapaxes(out_t, 0, 1)[..., None]


# speedup vs baseline: 1.6703x; 1.0045x over previous
import functools

import jax
import jax.numpy as jnp
from jax import lax
from jax.experimental import pallas as pl
from jax.experimental.pallas import tpu as pltpu
from jax.experimental.pallas import tpu_sc as plsc

_info = plsc.get_sparse_core_info()
_NC, _NS = _info.num_cores, _info.num_subcores
_NW = _NC * _NS

_K = 50


@functools.lru_cache(maxsize=None)
def _build(h: int, b: int, n_rows: int):
    assert b % (_NW * 8) == 0 and h % _K == 0
    cpw = b // _NW

    mesh = plsc.VectorSubcoreMesh(core_axis_name="c", subcore_axis_name="s")

    @functools.partial(
        pl.kernel,
        mesh=mesh,
        compiler_params=pltpu.CompilerParams(needs_layout_passes=False),
        out_type=jax.ShapeDtypeStruct((h, b), jnp.float32),
        scratch_types=[
            pltpu.VMEM((h, cpw), jnp.int32),
            pltpu.VMEM((h, cpw), jnp.float32),
            pltpu.VMEM_SHARED((n_rows,), jnp.float32),
            pltpu.SemaphoreType.DMA,
        ],
    )
    def gather_kernel(idx_hbm, tab_hbm, out_hbm, idx_v, rows_v, tab_sh,
                      sem_g):
        wid = lax.axis_index("s") * _NC + lax.axis_index("c")
        cb = wid * cpw

        @pl.when(lax.axis_index("s") == 0)
        def _stage():
            pltpu.sync_copy(tab_hbm, tab_sh)

        pltpu.sync_copy(idx_hbm.at[:, pl.ds(cb, cpw)], idx_v)
        plsc.subcore_barrier()

        def step(g, carry):
            j0 = g * _K
            gathers = [
                pltpu.async_copy(tab_sh.at[idx_v.at[j0 + j]],
                                 rows_v.at[j0 + j], sem_g)
                for j in range(_K)
            ]
            for c in gathers:
                c.wait()
            return carry

        lax.fori_loop(0, h // _K, step, 0, unroll=False)
        pltpu.sync_copy(rows_v, out_hbm.at[:, pl.ds(cb, cpw)])

    return gather_kernel


def kernel(input, table):
    b, h = input.shape
    idx_t = jnp.swapaxes(input, 0, 1)
    tab = jnp.squeeze(table, 1)
    out_t = _build(h, b, tab.shape[0])(idx_t, tab)
    return jnp.swapaxes(out_t, 0, 1)[..., None]
